# baseline (speedup 1.0000x reference)
import jax
import jax.numpy as jnp
from jax import lax
from jax.experimental import pallas as pl
from jax.experimental.pallas import tpu as pltpu
from jax.experimental.pallas import tpu_sc as plsc

_NC, _NS, _L = 2, 16, 16
_NW = _NC * _NS
_ROWS, _COLS = 1024, 16384
_RPW = _ROWS // _NW
_NBINS = 64
_BINS_PAD = 80
_CHUNK = 2048


_PREP_GRID = 8
_PREP_ROWS = _ROWS // _PREP_GRID


def _prep_body(x_ref, xl_ref, o_ref, acc_ref):
    i = pl.program_id(0)
    blk = x_ref[...]
    xl_ref[...] = blk.reshape(_PREP_ROWS * _COLS // 128, 128)
    bmin = jnp.min(blk)
    bmax = jnp.max(blk)
    rows = lax.broadcasted_iota(jnp.int32, (2, 128), 0)
    vals = jnp.where(rows == 0, bmin, bmax)

    @pl.when(i == 0)
    def _():
        acc_ref[...] = vals

    @pl.when(i != 0)
    def _():
        cur = acc_ref[...]
        acc_ref[...] = jnp.where(
            rows == 0, jnp.minimum(cur, vals), jnp.maximum(cur, vals))

    @pl.when(i == _PREP_GRID - 1)
    def _():
        o_ref[...] = acc_ref[...]


def _prep(x):
    return pl.pallas_call(
        _prep_body,
        grid=(_PREP_GRID,),
        in_specs=[pl.BlockSpec((_PREP_ROWS, _COLS), lambda i: (i, 0))],
        out_specs=[
            pl.BlockSpec((_PREP_ROWS * _COLS // 128, 128), lambda i: (i, 0)),
            pl.BlockSpec((2, 128), lambda i: (0, 0)),
        ],
        out_shape=[
            jax.ShapeDtypeStruct((_ROWS * _COLS // 128, 128), jnp.float32),
            jax.ShapeDtypeStruct((2, 128), jnp.float32),
        ],
        scratch_shapes=[pltpu.VMEM((2, 128), jnp.float32)],
    )(x)


_U = 8
_CEIL_C = 0.99999994


def _sc_body(x_hbm, stats_hbm, out_hbm, statsbuf, xbuf0, xbuf1, xbuf2,
             hist_s, hist_c, fusedbuf, sem0, sem1, sem2):
    wid = lax.axis_index("s") * _NC + lax.axis_index("c")
    base = wid * _RPW

    pltpu.sync_copy(stats_hbm.at[:, pl.ds(0, _L)], statsbuf)
    mn = statsbuf[0, :]
    mx = statsbuf[1, :]
    scale = 64.0 / (mx - mn + 1e-6)
    dconst = _CEIL_C - mn * scale

    zeros = jnp.zeros((_L,), jnp.float32)
    ones = jnp.ones((_L,), jnp.float32)
    lanes = lax.iota(jnp.int32, _L)

    def zbody(i, _):
        hist_s[pl.ds(i * _L, _L)] = zeros
        hist_c[pl.ds(i * _L, _L)] = zeros
        return 0

    lax.fori_loop(0, _BINS_PAD * _RPW // _L, zbody, 0)

    chunks = [(g, ch) for g in range(_RPW // _L)
              for ch in range(_COLS // _CHUNK)]
    bufs = (xbuf0, xbuf1, xbuf2)
    sems = (sem0, sem1, sem2)
    nbuf = len(bufs)

    def _start(k):
        g, ch = chunks[k]
        handles = []
        for r in range(_L):
            row = base + g * _L + r
            handles.append(pltpu.async_copy(
                x_hbm.at[pl.ds(row * _COLS + ch * _CHUNK, _CHUNK)],
                bufs[k % nbuf].at[r, pl.ds(0, _CHUNK)], sems[k % nbuf]))
        return handles

    inflight = {0: _start(0), 1: _start(1)}
    for k, (g, ch) in enumerate(chunks):
        buf = bufs[k % nbuf]
        if k + 2 < len(chunks):
            inflight[k + 2] = _start(k + 2)
        for h in inflight.pop(k):
            h.wait()
        rowvec = g * _L + lanes

        @plsc.parallel_loop(0, _CHUNK, 1, unroll=_U)
        def _(c):
            v = plsc.load_gather(
                buf, [lanes, jnp.full((_L,), c, jnp.int32)])
            t = v * scale + dconst
            flat = t.astype(jnp.int32) * _RPW + rowvec
            plsc.addupdate_scatter(hist_s, [flat], v)
            plsc.addupdate_scatter(hist_c, [flat], ones)

    for g in range(_RPW // _L):
        def fbody(i, acc):
            s = hist_s[pl.ds(i * _RPW + g * _L, _L)]
            c = hist_c[pl.ds(i * _RPW + g * _L, _L)]
            return acc + s / (c + 1e-6)

        acc = lax.fori_loop(0, _NBINS, fbody, zeros)
        fusedbuf[pl.ds(g * _L, _L)] = acc * (1.0 / 64.0)

    pltpu.sync_copy(fusedbuf, out_hbm.at[pl.ds(base, _RPW)])


_sc_mesh = plsc.VectorSubcoreMesh(
    core_axis_name="c", subcore_axis_name="s", num_cores=_NC,
    num_subcores=_NS)

_bucket_call = pl.kernel(
    _sc_body,
    out_type=jax.ShapeDtypeStruct((_ROWS,), jnp.float32),
    mesh=_sc_mesh,
    scratch_types=[
        pltpu.VMEM((2, _L), jnp.float32),
        pltpu.VMEM((_L, _CHUNK + 1), jnp.float32),
        pltpu.VMEM((_L, _CHUNK + 1), jnp.float32),
        pltpu.VMEM((_L, _CHUNK + 1), jnp.float32),
        pltpu.VMEM((_BINS_PAD * _RPW,), jnp.float32),
        pltpu.VMEM((_BINS_PAD * _RPW,), jnp.float32),
        pltpu.VMEM((_RPW,), jnp.float32),
        pltpu.SemaphoreType.DMA,
        pltpu.SemaphoreType.DMA,
        pltpu.SemaphoreType.DMA,
    ],
    compiler_params=pltpu.CompilerParams(
        use_tc_tiling_on_sc=False, needs_layout_passes=False),
)


def kernel(x, bin_edges):
    xlin, stats = _prep(x)
    fused = _bucket_call(jnp.reshape(xlin, (_ROWS * _COLS,)), stats)
    return fused[:, None]


# speedup vs baseline: 1.0091x; 1.0091x over previous
import jax
import jax.numpy as jnp
from jax import lax
from jax.experimental import pallas as pl
from jax.experimental.pallas import tpu as pltpu
from jax.experimental.pallas import tpu_sc as plsc

_NC, _NS, _L = 2, 16, 16
_NW = _NC * _NS
_ROWS, _COLS = 1024, 16384
_RPW = _ROWS // _NW
_NBINS = 64
_BINS_PAD = 80
_CHUNK = 2048


_PREP_GRID = 8
_PREP_ROWS = _ROWS // _PREP_GRID


def _prep_body(x_ref, xl_ref, o_ref, acc_ref):
    i = pl.program_id(0)
    blk = x_ref[...]
    xl_ref[...] = blk.reshape(_PREP_ROWS * _COLS // 128, 128)
    bmin = jnp.min(blk)
    bmax = jnp.max(blk)
    rows = lax.broadcasted_iota(jnp.int32, (2, 128), 0)
    vals = jnp.where(rows == 0, bmin, bmax)

    @pl.when(i == 0)
    def _():
        acc_ref[...] = vals

    @pl.when(i != 0)
    def _():
        cur = acc_ref[...]
        acc_ref[...] = jnp.where(
            rows == 0, jnp.minimum(cur, vals), jnp.maximum(cur, vals))

    @pl.when(i == _PREP_GRID - 1)
    def _():
        o_ref[...] = acc_ref[...]


def _prep(x):
    return pl.pallas_call(
        _prep_body,
        grid=(_PREP_GRID,),
        in_specs=[pl.BlockSpec((_PREP_ROWS, _COLS), lambda i: (i, 0))],
        out_specs=[
            pl.BlockSpec((_PREP_ROWS * _COLS // 128, 128), lambda i: (i, 0)),
            pl.BlockSpec((2, 128), lambda i: (0, 0)),
        ],
        out_shape=[
            jax.ShapeDtypeStruct((_ROWS * _COLS // 128, 128), jnp.float32),
            jax.ShapeDtypeStruct((2, 128), jnp.float32),
        ],
        scratch_shapes=[pltpu.VMEM((2, 128), jnp.float32)],
    )(x)


_U = 8
_CEIL_C = 0.99999994


def _sc_body(x_hbm, stats_hbm, out_hbm, statsbuf, xbuf0, xbuf1, hist_s,
             hist_c, fusedbuf, sem0, sem1):
    wid = lax.axis_index("s") * _NC + lax.axis_index("c")
    base = wid * _RPW

    pltpu.sync_copy(stats_hbm.at[:, pl.ds(0, _L)], statsbuf)
    mn = statsbuf[0, :]
    mx = statsbuf[1, :]
    scale = 64.0 / (mx - mn + 1e-6)
    dconst = _CEIL_C - mn * scale

    zeros = jnp.zeros((_L,), jnp.float32)
    ones = jnp.ones((_L,), jnp.float32)
    lanes = lax.iota(jnp.int32, _L)

    def zbody(i, _):
        hist_s[pl.ds(i * _L, _L)] = zeros
        hist_c[pl.ds(i * _L, _L)] = zeros
        return 0

    lax.fori_loop(0, _BINS_PAD * _RPW // _L, zbody, 0)

    chunks = [(g, ch) for g in range(_RPW // _L)
              for ch in range(_COLS // _CHUNK)]
    bufs = (xbuf0, xbuf1)
    sems = (sem0, sem1)
    nbuf = len(bufs)

    def _start(k):
        g, ch = chunks[k]
        handles = []
        for r in range(_L):
            row = base + g * _L + r
            handles.append(pltpu.async_copy(
                x_hbm.at[pl.ds(row * _COLS + ch * _CHUNK, _CHUNK)],
                bufs[k % nbuf].at[r, pl.ds(0, _CHUNK)], sems[k % nbuf]))
        return handles

    inflight = {0: _start(0)}
    for k, (g, ch) in enumerate(chunks):
        buf = bufs[k % nbuf]
        if k + 1 < len(chunks):
            inflight[k + 1] = _start(k + 1)
        for h in inflight.pop(k):
            h.wait()
        rowvec = g * _L + lanes

        @plsc.parallel_loop(0, _CHUNK, 1, unroll=_U)
        def _(c):
            v = plsc.load_gather(
                buf, [lanes, jnp.full((_L,), c, jnp.int32)])
            t = v * scale + dconst
            flat = t.astype(jnp.int32) * _RPW + rowvec
            plsc.addupdate_scatter(hist_s, [flat], v)
            plsc.addupdate_scatter(hist_c, [flat], ones)

    for g in range(_RPW // _L):
        def fbody(i, acc):
            s = hist_s[pl.ds(i * _RPW + g * _L, _L)]
            c = hist_c[pl.ds(i * _RPW + g * _L, _L)]
            return acc + s / (c + 1e-6)

        acc = lax.fori_loop(0, _NBINS, fbody, zeros)
        fusedbuf[pl.ds(g * _L, _L)] = acc * (1.0 / 64.0)

    pltpu.sync_copy(fusedbuf, out_hbm.at[pl.ds(base, _RPW)])


_sc_mesh = plsc.VectorSubcoreMesh(
    core_axis_name="c", subcore_axis_name="s", num_cores=_NC,
    num_subcores=_NS)

_bucket_call = pl.kernel(
    _sc_body,
    out_type=jax.ShapeDtypeStruct((_ROWS,), jnp.float32),
    mesh=_sc_mesh,
    scratch_types=[
        pltpu.VMEM((2, _L), jnp.float32),
        pltpu.VMEM((_L, _CHUNK + 1), jnp.float32),
        pltpu.VMEM((_L, _CHUNK + 1), jnp.float32),
        pltpu.VMEM((_BINS_PAD * _RPW,), jnp.float32),
        pltpu.VMEM((_BINS_PAD * _RPW,), jnp.float32),
        pltpu.VMEM((_RPW,), jnp.float32),
        pltpu.SemaphoreType.DMA,
        pltpu.SemaphoreType.DMA,
    ],
    compiler_params=pltpu.CompilerParams(
        use_tc_tiling_on_sc=False, needs_layout_passes=False),
)


def kernel(x, bin_edges):
    xlin, stats = _prep(x)
    fused = _bucket_call(jnp.reshape(xlin, (_ROWS * _COLS,)), stats)
    return fused[:, None]
